# baseline (speedup 1.0000x reference)
import functools

import jax
import jax.numpy as jnp
from jax import lax
from jax.experimental import pallas as pl
from jax.experimental.pallas import tpu as pltpu
from jax.experimental.pallas import tpu_sc as plsc


def _chunk_plan(per_w: int, main: int):
    taper = [48, 56, 96, 200]
    if per_w >= 2 * sum(taper) + main and (per_w - 2 * sum(taper)) % main == 0:
        n_main = (per_w - 2 * sum(taper)) // main
        return taper + [main] * n_main + taper[::-1]
    chunk = main
    while per_w % chunk != 0:
        chunk //= 2
    return [chunk] * (per_w // chunk)


def _make_gather(total: int, vocab: int, dim: int):
    info = plsc.get_sparse_core_info()
    nc, ns = info.num_cores, info.num_subcores
    nw = nc * ns
    assert total % nw == 0
    per_w = total // nw
    chunks = _chunk_plan(per_w, 400)
    starts = [0]
    for c in chunks:
        starts.append(starts[-1] + c)
    n_chunks = len(chunks)
    bufrows = max(chunks)
    nbuf = 4
    idx_split = min(400, per_w)

    mesh = plsc.VectorSubcoreMesh(core_axis_name="c", subcore_axis_name="s")

    @functools.partial(
        pl.kernel,
        out_type=jax.ShapeDtypeStruct((total, dim), jnp.float32),
        mesh=mesh,
        scratch_types=[
            pltpu.VMEM((per_w,), jnp.int32),
            [pltpu.VMEM((bufrows, dim), jnp.float32) for _ in range(nbuf)],
            [pltpu.SemaphoreType.DMA for _ in range(nbuf)],
            [pltpu.SemaphoreType.DMA for _ in range(nbuf)],
            pltpu.VMEM_SHARED((ns, 2, 128, dim), jnp.float32),
        ],
        compiler_params=pltpu.CompilerParams(use_tc_tiling_on_sc=False),
    )
    def gather(table_hbm, idx_hbm, out_hbm, idx_v, rows, gsems, wsems, stage):
        sid = lax.axis_index("s")
        wid = sid * nc + lax.axis_index("c")
        base = wid * per_w
        pltpu.sync_copy(idx_hbm.at[pl.ds(base, idx_split)],
                        idx_v.at[pl.ds(0, idx_split)])

        def issue_gather(c, b):
            pltpu.async_copy(
                table_hbm.at[idx_v.at[pl.ds(starts[c], chunks[c])]],
                rows[b].at[pl.ds(0, chunks[c])], gsems[b],
            )

        def wait_gather(c, b):
            pltpu.make_async_copy(
                table_hbm.at[idx_v.at[pl.ds(starts[c], chunks[c])]],
                rows[b].at[pl.ds(0, chunks[c])], gsems[b],
            ).wait()

        slot_state = [None, None]
        wcount = [0]

        def drain_slot(s):
            if slot_state[s] is not None:
                psz, pdst = slot_state[s]
                pltpu.make_async_copy(
                    stage.at[sid, s].at[pl.ds(0, psz)],
                    out_hbm.at[pl.ds(pdst, psz)], wsems[s],
                ).wait()
                slot_state[s] = None

        def write_chunk(c, b):
            off = 0
            rem = chunks[c]
            while rem:
                sz = min(128, rem)
                s = wcount[0] % 2
                drain_slot(s)
                pltpu.sync_copy(rows[b].at[pl.ds(off, sz)],
                                stage.at[sid, s].at[pl.ds(0, sz)])
                dst = base + starts[c] + off
                pltpu.async_copy(stage.at[sid, s].at[pl.ds(0, sz)],
                                 out_hbm.at[pl.ds(dst, sz)], wsems[s])
                slot_state[s] = (sz, dst)
                wcount[0] += 1
                off += sz
                rem -= sz

        n_first = 0
        while starts[n_first + 1] <= idx_split and n_first + 1 < n_chunks:
            n_first += 1
        n_first = min(n_first, nbuf)
        for c in range(n_first):
            issue_gather(c, c % nbuf)
        if idx_split < per_w:
            pltpu.sync_copy(idx_hbm.at[pl.ds(base + idx_split, per_w - idx_split)],
                            idx_v.at[pl.ds(idx_split, per_w - idx_split)])

        for i in range(n_chunks + nbuf - 1):
            if n_first <= i < n_chunks:
                issue_gather(i, i % nbuf)
            j = i - (nbuf - 1)
            if 0 <= j < n_chunks:
                b = j % nbuf
                wait_gather(j, b)
                write_chunk(j, b)
        drain_slot(0)
        drain_slot(1)

    return gather


def kernel(indices, table):
    b, l = indices.shape
    vocab, dim = table.shape
    flat = indices.reshape(b * l)
    gather = _make_gather(b * l, vocab, dim)
    out = gather(table, flat)
    return out.reshape(b, l, dim)
